# baseline (speedup 1.0000x reference)
import jax
import jax.numpy as jnp
from jax import lax
from jax.experimental import pallas as pl
from jax.experimental.pallas import tpu as pltpu
from jax.experimental.pallas import tpu_sc as plsc

B, S, D = 4, 4096, 1024
NC, NS = 2, 16
NW = NC * NS
ROWS_W = S // NW
C = 8
NCHUNK = ROWS_W // C
VECS = C * D // 16
VECS_PER_ROW = D // 16
UNROLL = 8
DEPTH = 2


def _sc_body(x_hbm, pos_hbm, out_hbm, xbuf, pbuf, xld, xst, pld):
    w = lax.axis_index("s") * NC + lax.axis_index("c")
    base = w * ROWS_W

    def start_pos(c):
        pltpu.async_copy(pos_hbm.at[pl.ds(base + c * C, C), :],
                         pbuf.at[c % DEPTH], pld.at[c % DEPTH])

    def wait_pos(c):
        pltpu.make_async_copy(pos_hbm.at[pl.ds(base + c * C, C), :],
                              pbuf.at[c % DEPTH], pld.at[c % DEPTH]).wait()

    def start_xload(c, b):
        slot = b * DEPTH + c % DEPTH
        pltpu.async_copy(x_hbm.at[pl.ds(b * S + base + c * C, C), :],
                         xbuf.at[slot], xld.at[slot])

    def wait_xload(c, b):
        slot = b * DEPTH + c % DEPTH
        pltpu.make_async_copy(x_hbm.at[pl.ds(b * S + base + c * C, C), :],
                              xbuf.at[slot], xld.at[slot]).wait()

    def start_store(c, b):
        slot = b * DEPTH + c % DEPTH
        pltpu.async_copy(xbuf.at[slot],
                         out_hbm.at[pl.ds(b * S + base + c * C, C), :],
                         xst.at[slot])

    def wait_store(c, b):
        slot = b * DEPTH + c % DEPTH
        pltpu.make_async_copy(
            xbuf.at[slot],
            out_hbm.at[pl.ds(b * S + base + c * C, C), :],
            xst.at[slot]).wait()

    start_pos(0)
    for b in range(B):
        start_xload(0, b)

    for c in range(NCHUNK):
        par = c % DEPTH
        if c + 1 < NCHUNK:
            start_pos(c + 1)
            for b in range(B):
                if c + 1 - DEPTH >= 0:
                    wait_store(c + 1 - DEPTH, b)
                start_xload(c + 1, b)
        wait_pos(c)
        for b in range(B):
            wait_xload(c, b)

        @plsc.parallel_loop(0, VECS, unroll=UNROLL)
        def _add(i):
            r = i // VECS_PER_ROW
            col = (i % VECS_PER_ROW) * 16
            sl = pl.ds(col, 16)
            p = pbuf[par, r, sl]
            for b in range(B):
                slot = b * DEPTH + par
                xbuf[slot, r, sl] = xbuf[slot, r, sl] + p

        for b in range(B):
            start_store(c, b)

    for c in range(max(0, NCHUNK - DEPTH), NCHUNK):
        for b in range(B):
            wait_store(c, b)


def kernel(x, pos_table):
    x2 = x.reshape(B * S, D)
    out = pl.kernel(
        _sc_body,
        out_type=jax.ShapeDtypeStruct((B * S, D), jnp.float32),
        mesh=plsc.VectorSubcoreMesh(core_axis_name="c", subcore_axis_name="s"),
        compiler_params=pltpu.CompilerParams(use_tc_tiling_on_sc=True),
        scratch_types=[
            pltpu.VMEM((DEPTH * B, C, D), jnp.float32),
            pltpu.VMEM((DEPTH, C, D), jnp.float32),
            pltpu.SemaphoreType.DMA((DEPTH * B,)),
            pltpu.SemaphoreType.DMA((DEPTH * B,)),
            pltpu.SemaphoreType.DMA((DEPTH,)),
        ],
    )(x2, pos_table)
    return out.reshape(B, S, D)


# speedup vs baseline: 1.0132x; 1.0132x over previous
import jax
import jax.numpy as jnp
from jax import lax
from jax.experimental import pallas as pl
from jax.experimental.pallas import tpu as pltpu
from jax.experimental.pallas import tpu_sc as plsc

B, S, D = 4, 4096, 1024
NC, NS = 2, 16
NW = NC * NS
ROWS_W = S // NW
C = 8
NCHUNK = ROWS_W // C
VECS = C * D // 16
VECS_PER_ROW = D // 16
UNROLL = 4
DEPTH = 2


def _sc_body(x_hbm, pos_hbm, out_hbm, xbuf, pbuf, xld, xst, pld):
    w = lax.axis_index("s") * NC + lax.axis_index("c")
    base = w * ROWS_W

    def start_pos(c):
        pltpu.async_copy(pos_hbm.at[pl.ds(base + c * C, C), :],
                         pbuf.at[c % DEPTH], pld.at[c % DEPTH])

    def wait_pos(c):
        pltpu.make_async_copy(pos_hbm.at[pl.ds(base + c * C, C), :],
                              pbuf.at[c % DEPTH], pld.at[c % DEPTH]).wait()

    def start_xload(c, b):
        slot = b * DEPTH + c % DEPTH
        pltpu.async_copy(x_hbm.at[pl.ds(b * S + base + c * C, C), :],
                         xbuf.at[slot], xld.at[slot])

    def wait_xload(c, b):
        slot = b * DEPTH + c % DEPTH
        pltpu.make_async_copy(x_hbm.at[pl.ds(b * S + base + c * C, C), :],
                              xbuf.at[slot], xld.at[slot]).wait()

    def start_store(c, b):
        slot = b * DEPTH + c % DEPTH
        pltpu.async_copy(xbuf.at[slot],
                         out_hbm.at[pl.ds(b * S + base + c * C, C), :],
                         xst.at[slot])

    def wait_store(c, b):
        slot = b * DEPTH + c % DEPTH
        pltpu.make_async_copy(
            xbuf.at[slot],
            out_hbm.at[pl.ds(b * S + base + c * C, C), :],
            xst.at[slot]).wait()

    start_pos(0)
    for b in range(B):
        start_xload(0, b)

    for c in range(NCHUNK):
        par = c % DEPTH
        if c + 1 < NCHUNK:
            start_pos(c + 1)
            for b in range(B):
                if c + 1 - DEPTH >= 0:
                    wait_store(c + 1 - DEPTH, b)
                start_xload(c + 1, b)
        wait_pos(c)
        for b in range(B):
            wait_xload(c, b)

        @plsc.parallel_loop(0, VECS, unroll=UNROLL)
        def _add(i):
            r = i // VECS_PER_ROW
            col = (i % VECS_PER_ROW) * 16
            sl = pl.ds(col, 16)
            p = pbuf[par, r, sl]
            for b in range(B):
                slot = b * DEPTH + par
                xbuf[slot, r, sl] = xbuf[slot, r, sl] + p

        for b in range(B):
            start_store(c, b)

    for c in range(max(0, NCHUNK - DEPTH), NCHUNK):
        for b in range(B):
            wait_store(c, b)


def kernel(x, pos_table):
    x2 = x.reshape(B * S, D)
    out = pl.kernel(
        _sc_body,
        out_type=jax.ShapeDtypeStruct((B * S, D), jnp.float32),
        mesh=plsc.VectorSubcoreMesh(core_axis_name="c", subcore_axis_name="s"),
        compiler_params=pltpu.CompilerParams(use_tc_tiling_on_sc=True),
        scratch_types=[
            pltpu.VMEM((DEPTH * B, C, D), jnp.float32),
            pltpu.VMEM((DEPTH, C, D), jnp.float32),
            pltpu.SemaphoreType.DMA((DEPTH * B,)),
            pltpu.SemaphoreType.DMA((DEPTH * B,)),
            pltpu.SemaphoreType.DMA((DEPTH,)),
        ],
    )(x2, pos_table)
    return out.reshape(B, S, D)
